# baseline (speedup 1.0000x reference)
import functools

import jax
import jax.numpy as jnp
from jax import lax
from jax.experimental import pallas as pl
from jax.experimental.pallas import tpu as pltpu
from jax.experimental.pallas import tpu_sc as plsc

B, L, S, D = 1024, 20, 12, 128
NP = B * L
N_NBR = NP * S
N_OTH = NP * 3 + B * 2 * L
NC, NS = 2, 16
NW = NC * NS
PPW = NP // NW
NBW = PPW * S
OPW = N_OTH // NW
CH = 128
NCH_N = NBW // CH
NCH_O = OPW // CH


def _sc_body(adj_flat, wgt_tbl, emb, urev, oth_idx,
             wgt_out, nbr_out, oth_out,
             urev_v, fidx, idxflat, wgtflat, othidx_v, rowbuf, sem):
    wid = lax.axis_index("s") * NC + lax.axis_index("c")
    pbase = wid * PPW
    pltpu.sync_copy(urev.at[pl.ds(pbase, PPW)], urev_v)

    def fidx_row(row, _):
        for t in range(8):
            k = (row * 8 + t) * 16 + lax.iota(jnp.int32, 16)
            q = k // S
            r = k - q * S
            u = plsc.load_gather(urev_v, [q])
            fidx[row, pl.ds(t * 16, 16)] = u * S + r
        return 0
    lax.fori_loop(0, NCH_N, fidx_row, 0)

    def elem_chunk(j, _):
        pltpu.async_copy(adj_flat.at[fidx.at[j]], idxflat.at[j], sem).wait()
        pltpu.async_copy(wgt_tbl.at[fidx.at[j]], wgtflat.at[j], sem).wait()
        return 0
    lax.fori_loop(0, NCH_N, elem_chunk, 0)
    pltpu.sync_copy(wgtflat, wgt_out.at[wid])

    def nbr_chunk(c, _):
        pltpu.async_copy(emb.at[idxflat.at[c]], rowbuf, sem).wait()
        pltpu.sync_copy(rowbuf, nbr_out.at[pl.ds(wid * NBW + c * CH, CH)])
        return 0
    lax.fori_loop(0, NCH_N, nbr_chunk, 0)

    obase = wid * OPW

    def oidx_row(j, _):
        pltpu.sync_copy(oth_idx.at[pl.ds(obase + j * CH, CH)], othidx_v.at[j])
        return 0
    lax.fori_loop(0, NCH_O, oidx_row, 0)

    def oth_chunk(c, _):
        pltpu.async_copy(emb.at[othidx_v.at[c]], rowbuf, sem).wait()
        pltpu.sync_copy(rowbuf, oth_out.at[pl.ds(obase + c * CH, CH)])
        return 0
    lax.fori_loop(0, NCH_O, oth_chunk, 0)


@jax.jit
def _sc_gather(adj_flat, wgt_tbl, emb, urev, oth_idx):
    mesh = plsc.VectorSubcoreMesh(core_axis_name="c", subcore_axis_name="s")
    return pl.kernel(
        _sc_body,
        out_type=(
            jax.ShapeDtypeStruct((NW, NCH_N, CH), jnp.float32),
            jax.ShapeDtypeStruct((N_NBR, D), jnp.float32),
            jax.ShapeDtypeStruct((N_OTH, D), jnp.float32),
        ),
        mesh=mesh,
        compiler_params=pltpu.CompilerParams(needs_layout_passes=False),
        scratch_types=[
            pltpu.VMEM((PPW,), jnp.int32),
            pltpu.VMEM((NCH_N, CH), jnp.int32),
            pltpu.VMEM((NCH_N, CH), jnp.int32),
            pltpu.VMEM((NCH_N, CH), jnp.float32),
            pltpu.VMEM((NCH_O, CH), jnp.int32),
            pltpu.VMEM((CH, D), jnp.float32),
            pltpu.SemaphoreType.DMA,
        ],
    )(adj_flat, wgt_tbl, emb, urev, oth_idx)


def _leaky(x):
    return jnp.where(x >= 0, x, 0.2 * x)


def _latt(h, adj, a0, a1, a2, a3):
    dn = (((1,), (1,)), ((), ()))
    e0 = _leaky(lax.dot_general(h * a0, h, dn))
    e1 = _leaky(lax.dot_general(h * a1, h, dn))
    e2 = _leaky(lax.dot_general(h * a2, h, dn))
    e3 = _leaky(lax.dot_general(h * a3, h, dn))
    al = jnp.full_like(e0, -9e15)
    al = jnp.where(adj == 1, e0, al)
    al = jnp.where(adj == 2, e1, al)
    al = jnp.where(adj == 3, e2, al)
    al = jnp.where(adj == 4, e3, al)
    al = al - jnp.max(al, axis=-1, keepdims=True)
    ex = jnp.exp(al)
    al = ex / jnp.sum(ex, axis=-1, keepdims=True)
    return jnp.dot(al, h)


def _tc_body(nbr_ref, wgt_ref, usess_ref, sess_ref, cats_ref, nods_ref,
             mask_ref, adji_ref, adjn_ref, ain_ref, aout_ref,
             w1a_ref, w1b_ref, w2_ref, w3a_ref, w3b_ref,
             la0_ref, la1_ref, la2_ref, la3_ref,
             ln0_ref, ln1_ref, ln2_ref, ln3_ref,
             win_ref, bin_ref, wout_ref, bout_ref, biah_ref, boah_ref,
             wihta_ref, wihtb_ref, bih_ref, whht_ref, bhh_ref,
             glob_ref, itm_ref, cat_ref, nod_ref):
    nbr = nbr_ref[0]
    wgtc = wgt_ref[0]
    us = usess_ref[0]
    sg = sess_ref[0]
    ct = cats_ref[0]
    nd = nods_ref[0]
    m2 = mask_ref[0]

    denom = jnp.maximum(jnp.sum(m2), 1.0)
    ave = jnp.dot(m2, sg) / denom

    xw = jnp.dot(nbr * ave, w1a_ref[...]) + wgtc * w1b_ref[...]
    sc = jnp.dot(_leaky(xw), w2_ref[...])
    e = jnp.exp(sc)
    row = lax.broadcasted_iota(jnp.int32, (L * S, L * S), 0) // S
    col = lax.broadcasted_iota(jnp.int32, (L * S, L * S), 1) // S
    mseg = (row == col).astype(jnp.float32)
    alpha = e / jnp.dot(mseg, e)
    prow = lax.broadcasted_iota(jnp.int32, (L, L * S), 0)
    pcol = lax.broadcasted_iota(jnp.int32, (L, L * S), 1) // S
    psel = (prow == pcol).astype(jnp.float32)
    agg = jnp.dot(psel, alpha * nbr)
    glob = jax.nn.relu(jnp.dot(us, w3a_ref[...]) + jnp.dot(agg, w3b_ref[...]))
    glob_ref[0] = glob

    itm_ref[0] = _latt(us, adji_ref[0], la0_ref[...], la1_ref[...],
                       la2_ref[...], la3_ref[...])

    hv = ct
    hi = jnp.dot(ain_ref[0], jnp.dot(hv, win_ref[...]) + bin_ref[...]) \
        + biah_ref[...]
    ho = jnp.dot(aout_ref[0], jnp.dot(hv, wout_ref[...]) + bout_ref[...]) \
        + boah_ref[...]
    gi = jnp.dot(hi, wihta_ref[...]) + jnp.dot(ho, wihtb_ref[...]) \
        + bih_ref[...]
    gh = jnp.dot(hv, whht_ref[...]) + bhh_ref[...]
    r = jax.nn.sigmoid(gi[:, :D] + gh[:, :D])
    z = jax.nn.sigmoid(gi[:, D:2 * D] + gh[:, D:2 * D])
    n = jnp.tanh(gi[:, 2 * D:] + r * gh[:, 2 * D:])
    cat_ref[0] = n + z * (hv - n)

    nod = _latt(nd, adjn_ref[0], ln0_ref[...], ln1_ref[...],
                ln2_ref[...], ln3_ref[...])
    nod = _latt(nod, adjn_ref[0], ln0_ref[...], ln1_ref[...],
                ln2_ref[...], ln3_ref[...])
    nod_ref[0] = nod


def _full(shape):
    return pl.BlockSpec(shape, lambda i: (0,) * len(shape))


@jax.jit
def _tc_dense(nbr, wgt3, usess, sessg, catsg, nodsg, maskf, adji, adjn,
              ain, aout, w1a, w1b, w2, w3a, w3b,
              la0, la1, la2, la3, ln0, ln1, ln2, ln3,
              win, bin_, wout, bout, biah, boah,
              wihta, wihtb, bih, whht, bhh):
    def bs(shape):
        return pl.BlockSpec((1,) + shape, lambda i: (i,) + (0,) * len(shape))
    f32 = jnp.float32
    out_shapes = (
        jax.ShapeDtypeStruct((B, L, D), f32),
        jax.ShapeDtypeStruct((B, L, D), f32),
        jax.ShapeDtypeStruct((B, L, D), f32),
        jax.ShapeDtypeStruct((B, 2 * L, D), f32),
    )
    in_specs = [
        bs((L * S, D)), bs((L * S, 1)), bs((L, D)), bs((L, D)), bs((L, D)),
        bs((2 * L, D)), bs((1, L)), bs((L, L)), bs((2 * L, 2 * L)),
        bs((L, L)), bs((L, L)),
        _full((D, D)), _full((1, D)), _full((D, 1)), _full((D, D)),
        _full((D, D)),
        _full((1, D)), _full((1, D)), _full((1, D)), _full((1, D)),
        _full((1, D)), _full((1, D)), _full((1, D)), _full((1, D)),
        _full((D, D)), _full((1, D)), _full((D, D)), _full((1, D)),
        _full((1, D)), _full((1, D)),
        _full((D, 3 * D)), _full((D, 3 * D)), _full((1, 3 * D)),
        _full((D, 3 * D)), _full((1, 3 * D)),
    ]
    out_specs = (bs((L, D)), bs((L, D)), bs((L, D)), bs((2 * L, D)))
    return pl.pallas_call(
        _tc_body,
        grid=(B,),
        in_specs=in_specs,
        out_specs=out_specs,
        out_shape=out_shapes,
    )(nbr, wgt3, usess, sessg, catsg, nodsg, maskf, adji, adjn, ain, aout,
      w1a, w1b, w2, w3a, w3b, la0, la1, la2, la3, ln0, ln1, ln2, ln3,
      win, bin_, wout, bout, biah, boah, wihta, wihtb, bih, whht, bhh)


def kernel(sess_idxs, global_adj_items, global_adj_wgts, urev_sess_itms,
           local_adj_itms, mask, rev_sess_itms, urev_sess_cats,
           local_adj_cats, urev_sess_nods, local_adj_nods, emb,
           la_a0, la_a1, la_a2, la_a3, lan_a0, lan_a1, lan_a2, lan_a3,
           W_in, b_in, W_out, b_out, b_iah, b_oah, w_ih, b_ih, w_hh, b_hh,
           ga_w1, ga_w2, ga_w3):
    i32 = jnp.int32
    adj_flat = global_adj_items.astype(i32).reshape(-1)
    wgt_tbl = global_adj_wgts.reshape(-1)
    urev = urev_sess_itms.reshape(-1).astype(i32)
    oth_idx = jnp.concatenate([
        urev, rev_sess_itms.reshape(-1).astype(i32),
        urev_sess_cats.reshape(-1).astype(i32),
        urev_sess_nods.reshape(-1).astype(i32)])

    wgt_flat, nbr_rows, oth_rows = _sc_gather(adj_flat, wgt_tbl, emb, urev,
                                              oth_idx)

    usess = oth_rows[:NP].reshape(B, L, D)
    sessg = oth_rows[NP:2 * NP].reshape(B, L, D)
    catsg = oth_rows[2 * NP:3 * NP].reshape(B, L, D)
    nodsg = oth_rows[3 * NP:].reshape(B, 2 * L, D)
    nbr = nbr_rows.reshape(B, L * S, D)
    wgt3 = wgt_flat.reshape(B, L * S, 1)
    maskf = mask.astype(jnp.float32).reshape(B, 1, L)
    ain = local_adj_cats[:, :, :L]
    aout = local_adj_cats[:, :, L:]
    wihT = w_ih.T
    w1a = ga_w1[:D]
    w1b = ga_w1[D:].reshape(1, D)

    return _tc_dense(
        nbr, wgt3, usess, sessg, catsg, nodsg, maskf,
        local_adj_itms.astype(i32), local_adj_nods.astype(i32), ain, aout,
        w1a, w1b, ga_w2, ga_w3[:D], ga_w3[D:],
        la_a0.reshape(1, D), la_a1.reshape(1, D), la_a2.reshape(1, D),
        la_a3.reshape(1, D), lan_a0.reshape(1, D), lan_a1.reshape(1, D),
        lan_a2.reshape(1, D), lan_a3.reshape(1, D),
        W_in, b_in.reshape(1, D), W_out, b_out.reshape(1, D),
        b_iah.reshape(1, D), b_oah.reshape(1, D),
        wihT[:D], wihT[D:], b_ih.reshape(1, 3 * D),
        w_hh.T, b_hh.reshape(1, 3 * D))


# speedup vs baseline: 2.4381x; 2.4381x over previous
import functools

import jax
import jax.numpy as jnp
from jax import lax
from jax.experimental import pallas as pl
from jax.experimental.pallas import tpu as pltpu
from jax.experimental.pallas import tpu_sc as plsc

B, L, S, D = 1024, 20, 12, 128
NP = B * L
N_NBR = NP * S
N_OTH = NP * 3 + B * 2 * L
NC, NS = 2, 16
NW = NC * NS
PPW = NP // NW
NBW = PPW * S
OPW = N_OTH // NW
CH = 128
NCH_N = NBW // CH
NCH_O = OPW // CH


def _sc_body(adj_flat, wgt_tbl, emb, urev, oth_idx,
             wgt_out, nbr_out, oth_out,
             urev_v, fidx, idxflat, wgtflat, othidx_v, rowbuf, sem):
    wid = lax.axis_index("s") * NC + lax.axis_index("c")
    pbase = wid * PPW
    pltpu.sync_copy(urev.at[pl.ds(pbase, PPW)], urev_v)

    def fidx_row(row, _):
        for t in range(8):
            k = (row * 8 + t) * 16 + lax.iota(jnp.int32, 16)
            q = k // S
            r = k - q * S
            u = plsc.load_gather(urev_v, [q])
            fidx[row, pl.ds(t * 16, 16)] = u * S + r
        return 0
    lax.fori_loop(0, NCH_N, fidx_row, 0)

    def elem_chunk(j, _):
        pltpu.async_copy(adj_flat.at[fidx.at[j]], idxflat.at[j], sem).wait()
        pltpu.async_copy(wgt_tbl.at[fidx.at[j]], wgtflat.at[j], sem).wait()
        return 0
    lax.fori_loop(0, NCH_N, elem_chunk, 0)
    pltpu.sync_copy(wgtflat, wgt_out.at[wid])

    def nbr_chunk(c, _):
        pltpu.async_copy(emb.at[idxflat.at[c]], rowbuf, sem).wait()
        pltpu.sync_copy(rowbuf, nbr_out.at[pl.ds(wid * NBW + c * CH, CH)])
        return 0
    lax.fori_loop(0, NCH_N, nbr_chunk, 0)

    obase = wid * OPW

    def oidx_row(j, _):
        pltpu.sync_copy(oth_idx.at[pl.ds(obase + j * CH, CH)], othidx_v.at[j])
        return 0
    lax.fori_loop(0, NCH_O, oidx_row, 0)

    def oth_chunk(c, _):
        pltpu.async_copy(emb.at[othidx_v.at[c]], rowbuf, sem).wait()
        pltpu.sync_copy(rowbuf, oth_out.at[pl.ds(obase + c * CH, CH)])
        return 0
    lax.fori_loop(0, NCH_O, oth_chunk, 0)


@jax.jit
def _sc_gather(adj_flat, wgt_tbl, emb, urev, oth_idx):
    mesh = plsc.VectorSubcoreMesh(core_axis_name="c", subcore_axis_name="s")
    return pl.kernel(
        _sc_body,
        out_type=(
            jax.ShapeDtypeStruct((NW, NCH_N, CH), jnp.float32),
            jax.ShapeDtypeStruct((N_NBR, D), jnp.float32),
            jax.ShapeDtypeStruct((N_OTH, D), jnp.float32),
        ),
        mesh=mesh,
        compiler_params=pltpu.CompilerParams(needs_layout_passes=False),
        scratch_types=[
            pltpu.VMEM((PPW,), jnp.int32),
            pltpu.VMEM((NCH_N, CH), jnp.int32),
            pltpu.VMEM((NCH_N, CH), jnp.int32),
            pltpu.VMEM((NCH_N, CH), jnp.float32),
            pltpu.VMEM((NCH_O, CH), jnp.int32),
            pltpu.VMEM((CH, D), jnp.float32),
            pltpu.SemaphoreType.DMA,
        ],
    )(adj_flat, wgt_tbl, emb, urev, oth_idx)


def _leaky(x):
    return jnp.where(x >= 0, x, 0.2 * x)


BB = 8


def _blockdiag(blocks, n, dtype):
    rows = []
    for b, blk in enumerate(blocks):
        parts = []
        if b > 0:
            parts.append(jnp.zeros((n, b * n), dtype))
        parts.append(blk)
        if b < len(blocks) - 1:
            parts.append(jnp.zeros((n, (len(blocks) - 1 - b) * n), dtype))
        rows.append(jnp.concatenate(parts, axis=1))
    return jnp.concatenate(rows, axis=0)


def _latt(H, adj_ref, a0, a1, a2, a3, n):
    N = BB * n
    dn = (((1,), (1,)), ((), ()))
    e0 = _leaky(lax.dot_general(H * a0, H, dn))
    e1 = _leaky(lax.dot_general(H * a1, H, dn))
    e2 = _leaky(lax.dot_general(H * a2, H, dn))
    e3 = _leaky(lax.dot_general(H * a3, H, dn))
    ablocks = []
    for b in range(BB):
        sl = (slice(b * n, (b + 1) * n),) * 2
        adj = adj_ref[0, b]
        al = jnp.full((n, n), -9e15, jnp.float32)
        al = jnp.where(adj == 1, e0[sl], al)
        al = jnp.where(adj == 2, e1[sl], al)
        al = jnp.where(adj == 3, e2[sl], al)
        al = jnp.where(adj == 4, e3[sl], al)
        al = al - jnp.max(al, axis=-1, keepdims=True)
        ex = jnp.exp(al)
        ablocks.append(ex / jnp.sum(ex, axis=-1, keepdims=True))
    return jnp.dot(_blockdiag(ablocks, n, jnp.float32), H)


def _tc_body(nbr_ref, wgt_ref, usess_ref, sess_ref, cats_ref, nods_ref,
             mask_ref, adji_ref, adjn_ref, ain_ref, aout_ref,
             w1a_ref, w1b_ref, w2_ref, w3a_ref, w3b_ref,
             la0_ref, la1_ref, la2_ref, la3_ref,
             ln0_ref, ln1_ref, ln2_ref, ln3_ref,
             win_ref, bin_ref, wout_ref, bout_ref, biah_ref, boah_ref,
             wihta_ref, wihtb_ref, bih_ref, whht_ref, bhh_ref,
             glob_ref, itm_ref, cat_ref, nod_ref):
    NR = BB * L * S
    NU = BB * L
    nbr = nbr_ref[0]
    wgtc = wgt_ref[0]
    us = usess_ref[0]
    sg = sess_ref[0]
    ct = cats_ref[0]
    nd = nods_ref[0]
    m2 = mask_ref[0]

    brow = lax.broadcasted_iota(jnp.int32, (BB, NU), 0)
    bcol = lax.broadcasted_iota(jnp.int32, (BB, NU), 1) // L
    mbd = jnp.where(brow == bcol, m2, 0.0)
    denom = jnp.maximum(jnp.sum(mbd, axis=-1, keepdims=True), 1.0)
    ave = jnp.dot(mbd, sg) / denom
    ind = (lax.broadcasted_iota(jnp.int32, (NR, BB), 0) // (L * S)
           == lax.broadcasted_iota(jnp.int32, (NR, BB), 1))
    avebig = jnp.dot(ind.astype(jnp.float32), ave)

    xw = jnp.dot(nbr * avebig, w1a_ref[...]) + wgtc * w1b_ref[...]
    sc = jnp.dot(_leaky(xw), w2_ref[...])
    e = jnp.exp(sc)
    seg = (lax.broadcasted_iota(jnp.int32, (NU, NR), 0)
           == lax.broadcasted_iota(jnp.int32, (NU, NR), 1) // S)
    seg = seg.astype(jnp.float32)
    segt = (lax.broadcasted_iota(jnp.int32, (NR, NU), 0) // S
            == lax.broadcasted_iota(jnp.int32, (NR, NU), 1))
    segsum = jnp.dot(seg, e)
    dsum = jnp.dot(segt.astype(jnp.float32), segsum)
    agg = jnp.dot(seg, (e / dsum) * nbr)
    glob = jax.nn.relu(jnp.dot(us, w3a_ref[...]) + jnp.dot(agg, w3b_ref[...]))
    glob_ref[0] = glob

    itm_ref[0] = _latt(us, adji_ref, la0_ref[...], la1_ref[...],
                       la2_ref[...], la3_ref[...], L)

    ain = _blockdiag([ain_ref[0, b] for b in range(BB)], L, jnp.float32)
    aout = _blockdiag([aout_ref[0, b] for b in range(BB)], L, jnp.float32)
    hv = ct
    hi = jnp.dot(ain, jnp.dot(hv, win_ref[...]) + bin_ref[...]) \
        + biah_ref[...]
    ho = jnp.dot(aout, jnp.dot(hv, wout_ref[...]) + bout_ref[...]) \
        + boah_ref[...]
    gi = jnp.dot(hi, wihta_ref[...]) + jnp.dot(ho, wihtb_ref[...]) \
        + bih_ref[...]
    gh = jnp.dot(hv, whht_ref[...]) + bhh_ref[...]
    r = jax.nn.sigmoid(gi[:, :D] + gh[:, :D])
    z = jax.nn.sigmoid(gi[:, D:2 * D] + gh[:, D:2 * D])
    n = jnp.tanh(gi[:, 2 * D:] + r * gh[:, 2 * D:])
    cat_ref[0] = n + z * (hv - n)

    nod = _latt(nd, adjn_ref, ln0_ref[...], ln1_ref[...],
                ln2_ref[...], ln3_ref[...], 2 * L)
    nod = _latt(nod, adjn_ref, ln0_ref[...], ln1_ref[...],
                ln2_ref[...], ln3_ref[...], 2 * L)
    nod_ref[0] = nod


def _full(shape):
    return pl.BlockSpec(shape, lambda i: (0,) * len(shape))


@jax.jit
def _tc_dense(nbr, wgt3, usess, sessg, catsg, nodsg, maskf, adji, adjn,
              ain, aout, w1a, w1b, w2, w3a, w3b,
              la0, la1, la2, la3, ln0, ln1, ln2, ln3,
              win, bin_, wout, bout, biah, boah,
              wihta, wihtb, bih, whht, bhh):
    def bs(shape):
        return pl.BlockSpec((1,) + shape, lambda i: (i,) + (0,) * len(shape))
    f32 = jnp.float32
    G = B // BB
    out_shapes = (
        jax.ShapeDtypeStruct((G, BB * L, D), f32),
        jax.ShapeDtypeStruct((G, BB * L, D), f32),
        jax.ShapeDtypeStruct((G, BB * L, D), f32),
        jax.ShapeDtypeStruct((G, BB * 2 * L, D), f32),
    )
    in_specs = [
        bs((BB * L * S, D)), bs((BB * L * S, 1)), bs((BB * L, D)),
        bs((BB * L, D)), bs((BB * L, D)), bs((BB * 2 * L, D)),
        bs((1, BB * L)), bs((BB, L, L)), bs((BB, 2 * L, 2 * L)),
        bs((BB, L, L)), bs((BB, L, L)),
        _full((D, D)), _full((1, D)), _full((D, 1)), _full((D, D)),
        _full((D, D)),
        _full((1, D)), _full((1, D)), _full((1, D)), _full((1, D)),
        _full((1, D)), _full((1, D)), _full((1, D)), _full((1, D)),
        _full((D, D)), _full((1, D)), _full((D, D)), _full((1, D)),
        _full((1, D)), _full((1, D)),
        _full((D, 3 * D)), _full((D, 3 * D)), _full((1, 3 * D)),
        _full((D, 3 * D)), _full((1, 3 * D)),
    ]
    out_specs = (bs((BB * L, D)), bs((BB * L, D)), bs((BB * L, D)),
                 bs((BB * 2 * L, D)))
    return pl.pallas_call(
        _tc_body,
        grid=(G,),
        in_specs=in_specs,
        out_specs=out_specs,
        out_shape=out_shapes,
    )(nbr, wgt3, usess, sessg, catsg, nodsg, maskf, adji, adjn, ain, aout,
      w1a, w1b, w2, w3a, w3b, la0, la1, la2, la3, ln0, ln1, ln2, ln3,
      win, bin_, wout, bout, biah, boah, wihta, wihtb, bih, whht, bhh)


def kernel(sess_idxs, global_adj_items, global_adj_wgts, urev_sess_itms,
           local_adj_itms, mask, rev_sess_itms, urev_sess_cats,
           local_adj_cats, urev_sess_nods, local_adj_nods, emb,
           la_a0, la_a1, la_a2, la_a3, lan_a0, lan_a1, lan_a2, lan_a3,
           W_in, b_in, W_out, b_out, b_iah, b_oah, w_ih, b_ih, w_hh, b_hh,
           ga_w1, ga_w2, ga_w3):
    i32 = jnp.int32
    adj_flat = global_adj_items.astype(i32).reshape(-1)
    wgt_tbl = global_adj_wgts.reshape(-1)
    urev = urev_sess_itms.reshape(-1).astype(i32)
    oth_idx = jnp.concatenate([
        urev, rev_sess_itms.reshape(-1).astype(i32),
        urev_sess_cats.reshape(-1).astype(i32),
        urev_sess_nods.reshape(-1).astype(i32)])

    wgt_flat, nbr_rows, oth_rows = _sc_gather(adj_flat, wgt_tbl, emb, urev,
                                              oth_idx)

    G = B // BB
    usess = oth_rows[:NP].reshape(G, BB * L, D)
    sessg = oth_rows[NP:2 * NP].reshape(G, BB * L, D)
    catsg = oth_rows[2 * NP:3 * NP].reshape(G, BB * L, D)
    nodsg = oth_rows[3 * NP:].reshape(G, BB * 2 * L, D)
    nbr = nbr_rows.reshape(G, BB * L * S, D)
    wgt3 = wgt_flat.reshape(G, BB * L * S, 1)
    maskf = mask.astype(jnp.float32).reshape(G, 1, BB * L)
    ain = local_adj_cats[:, :, :L].reshape(G, BB, L, L)
    aout = local_adj_cats[:, :, L:].reshape(G, BB, L, L)
    wihT = w_ih.T
    w1a = ga_w1[:D]
    w1b = ga_w1[D:].reshape(1, D)

    glob, itm, cat, nod = _tc_dense(
        nbr, wgt3, usess, sessg, catsg, nodsg, maskf,
        local_adj_itms.astype(i32).reshape(G, BB, L, L),
        local_adj_nods.astype(i32).reshape(G, BB, 2 * L, 2 * L), ain, aout,
        w1a, w1b, ga_w2, ga_w3[:D], ga_w3[D:],
        la_a0.reshape(1, D), la_a1.reshape(1, D), la_a2.reshape(1, D),
        la_a3.reshape(1, D), lan_a0.reshape(1, D), lan_a1.reshape(1, D),
        lan_a2.reshape(1, D), lan_a3.reshape(1, D),
        W_in, b_in.reshape(1, D), W_out, b_out.reshape(1, D),
        b_iah.reshape(1, D), b_oah.reshape(1, D),
        wihT[:D], wihT[D:], b_ih.reshape(1, 3 * D),
        w_hh.T, b_hh.reshape(1, 3 * D))
    return (glob.reshape(B, L, D), itm.reshape(B, L, D),
            cat.reshape(B, L, D), nod.reshape(B, 2 * L, D))


# speedup vs baseline: 2.6524x; 1.0879x over previous
import functools

import jax
import jax.numpy as jnp
from jax import lax
from jax.experimental import pallas as pl
from jax.experimental.pallas import tpu as pltpu
from jax.experimental.pallas import tpu_sc as plsc

B, L, S, D = 1024, 20, 12, 128
NP = B * L
N_NBR = NP * S
N_OTH = NP * 3 + B * 2 * L
NC, NS = 2, 16
NW = NC * NS
PPW = NP // NW
NBW = PPW * S
OPW = N_OTH // NW
CH = 128
NCH_N = NBW // CH
NCH_O = OPW // CH


def _sc_body(adj_flat, wgt_tbl, emb, urev, oth_idx,
             wgt_out, nbr_out, oth_out,
             urev_v, fidx, idxflat, wgtflat, othidx_v, rowbuf, sem):
    wid = lax.axis_index("s") * NC + lax.axis_index("c")
    pbase = wid * PPW
    pltpu.sync_copy(urev.at[pl.ds(pbase, PPW)], urev_v)

    def fidx_row(row, _):
        for t in range(8):
            k = (row * 8 + t) * 16 + lax.iota(jnp.int32, 16)
            q = k // S
            r = k - q * S
            u = plsc.load_gather(urev_v, [q])
            fidx[row, pl.ds(t * 16, 16)] = u * S + r
        return 0
    lax.fori_loop(0, NCH_N, fidx_row, 0)

    def elem_chunk(j, _):
        pltpu.async_copy(adj_flat.at[fidx.at[j]], idxflat.at[j], sem).wait()
        pltpu.async_copy(wgt_tbl.at[fidx.at[j]], wgtflat.at[j], sem).wait()
        return 0
    lax.fori_loop(0, NCH_N, elem_chunk, 0)
    pltpu.sync_copy(wgtflat, wgt_out.at[wid])

    def nbr_chunk(c, _):
        pltpu.async_copy(emb.at[idxflat.at[c]], rowbuf, sem).wait()
        pltpu.sync_copy(rowbuf, nbr_out.at[pl.ds(wid * NBW + c * CH, CH)])
        return 0
    lax.fori_loop(0, NCH_N, nbr_chunk, 0)

    obase = wid * OPW

    def oidx_row(j, _):
        pltpu.sync_copy(oth_idx.at[pl.ds(obase + j * CH, CH)], othidx_v.at[j])
        return 0
    lax.fori_loop(0, NCH_O, oidx_row, 0)

    def oth_chunk(c, _):
        pltpu.async_copy(emb.at[othidx_v.at[c]], rowbuf, sem).wait()
        pltpu.sync_copy(rowbuf, oth_out.at[pl.ds(obase + c * CH, CH)])
        return 0
    lax.fori_loop(0, NCH_O, oth_chunk, 0)


@jax.jit
def _sc_gather(adj_flat, wgt_tbl, emb, urev, oth_idx):
    mesh = plsc.VectorSubcoreMesh(core_axis_name="c", subcore_axis_name="s")
    return pl.kernel(
        _sc_body,
        out_type=(
            jax.ShapeDtypeStruct((NW, NCH_N, CH), jnp.float32),
            jax.ShapeDtypeStruct((N_NBR, D), jnp.float32),
            jax.ShapeDtypeStruct((N_OTH, D), jnp.float32),
        ),
        mesh=mesh,
        compiler_params=pltpu.CompilerParams(needs_layout_passes=False),
        scratch_types=[
            pltpu.VMEM((PPW,), jnp.int32),
            pltpu.VMEM((NCH_N, CH), jnp.int32),
            pltpu.VMEM((NCH_N, CH), jnp.int32),
            pltpu.VMEM((NCH_N, CH), jnp.float32),
            pltpu.VMEM((NCH_O, CH), jnp.int32),
            pltpu.VMEM((CH, D), jnp.float32),
            pltpu.SemaphoreType.DMA,
        ],
    )(adj_flat, wgt_tbl, emb, urev, oth_idx)


def _leaky(x):
    return jnp.where(x >= 0, x, 0.2 * x)


BB = 8


def _blockdiag(blocks, n, dtype):
    rows = []
    for b, blk in enumerate(blocks):
        parts = []
        if b > 0:
            parts.append(jnp.zeros((n, b * n), dtype))
        parts.append(blk)
        if b < len(blocks) - 1:
            parts.append(jnp.zeros((n, (len(blocks) - 1 - b) * n), dtype))
        rows.append(jnp.concatenate(parts, axis=1))
    return jnp.concatenate(rows, axis=0)


def _latt(H, adj_ref, a0, a1, a2, a3, n):
    N = BB * n
    dn = (((1,), (1,)), ((), ()))
    e0 = _leaky(lax.dot_general(H * a0, H, dn))
    e1 = _leaky(lax.dot_general(H * a1, H, dn))
    e2 = _leaky(lax.dot_general(H * a2, H, dn))
    e3 = _leaky(lax.dot_general(H * a3, H, dn))
    ablocks = []
    for b in range(BB):
        sl = (slice(b * n, (b + 1) * n),) * 2
        adj = adj_ref[0, b]
        al = jnp.full((n, n), -9e15, jnp.float32)
        al = jnp.where(adj == 1, e0[sl], al)
        al = jnp.where(adj == 2, e1[sl], al)
        al = jnp.where(adj == 3, e2[sl], al)
        al = jnp.where(adj == 4, e3[sl], al)
        al = al - jnp.max(al, axis=-1, keepdims=True)
        ablocks.append(jnp.exp(al))
    ex = _blockdiag(ablocks, n, jnp.float32)
    return jnp.dot(ex, H) / jnp.sum(ex, axis=-1, keepdims=True)


def _tc_body(nbr_ref, wgt_ref, usess_ref, sess_ref, cats_ref, nods_ref,
             mask_ref, adji_ref, adjn_ref, ain_ref, aout_ref,
             seg_ref, ind_ref, bsel_ref,
             w1a_ref, w1b_ref, w2_ref, w3a_ref, w3b_ref,
             la0_ref, la1_ref, la2_ref, la3_ref,
             ln0_ref, ln1_ref, ln2_ref, ln3_ref,
             win_ref, bin_ref, wout_ref, bout_ref, biah_ref, boah_ref,
             wihta_ref, wihtb_ref, bih_ref, whht_ref, bhh_ref,
             glob_ref, itm_ref, cat_ref, nod_ref):
    nbr = nbr_ref[0]
    wgtc = wgt_ref[0]
    us = usess_ref[0]
    sg = sess_ref[0]
    ct = cats_ref[0]
    nd = nods_ref[0]
    m2 = mask_ref[0]
    seg = seg_ref[...]
    ind = ind_ref[...]
    bsel = bsel_ref[...]

    mbd = bsel * m2
    denom = jnp.maximum(jnp.sum(mbd, axis=-1, keepdims=True), 1.0)
    ave = jnp.dot(mbd, sg) / denom
    avebig = jnp.dot(ind, ave)

    xw = jnp.dot(nbr * avebig, w1a_ref[...]) + wgtc * w1b_ref[...]
    sc = jnp.dot(_leaky(xw), w2_ref[...])
    e = jnp.exp(sc)
    segsum = jnp.dot(seg, e)
    agg = jnp.dot(seg, e * nbr) / segsum
    glob = jax.nn.relu(jnp.dot(us, w3a_ref[...]) + jnp.dot(agg, w3b_ref[...]))
    glob_ref[0] = glob

    itm_ref[0] = _latt(us, adji_ref, la0_ref[...], la1_ref[...],
                       la2_ref[...], la3_ref[...], L)

    ain = _blockdiag([ain_ref[0, b] for b in range(BB)], L, jnp.float32)
    aout = _blockdiag([aout_ref[0, b] for b in range(BB)], L, jnp.float32)
    hv = ct
    hi = jnp.dot(ain, jnp.dot(hv, win_ref[...]) + bin_ref[...]) \
        + biah_ref[...]
    ho = jnp.dot(aout, jnp.dot(hv, wout_ref[...]) + bout_ref[...]) \
        + boah_ref[...]
    gi = jnp.dot(hi, wihta_ref[...]) + jnp.dot(ho, wihtb_ref[...]) \
        + bih_ref[...]
    gh = jnp.dot(hv, whht_ref[...]) + bhh_ref[...]
    r = jax.nn.sigmoid(gi[:, :D] + gh[:, :D])
    z = jax.nn.sigmoid(gi[:, D:2 * D] + gh[:, D:2 * D])
    n = jnp.tanh(gi[:, 2 * D:] + r * gh[:, 2 * D:])
    cat_ref[0] = n + z * (hv - n)

    nod = _latt(nd, adjn_ref, ln0_ref[...], ln1_ref[...],
                ln2_ref[...], ln3_ref[...], 2 * L)
    nod = _latt(nod, adjn_ref, ln0_ref[...], ln1_ref[...],
                ln2_ref[...], ln3_ref[...], 2 * L)
    nod_ref[0] = nod


def _full(shape):
    return pl.BlockSpec(shape, lambda i: (0,) * len(shape))


@jax.jit
def _tc_dense(nbr, wgt3, usess, sessg, catsg, nodsg, maskf, adji, adjn,
              ain, aout, seg, ind, bsel, w1a, w1b, w2, w3a, w3b,
              la0, la1, la2, la3, ln0, ln1, ln2, ln3,
              win, bin_, wout, bout, biah, boah,
              wihta, wihtb, bih, whht, bhh):
    def bs(shape):
        return pl.BlockSpec((1,) + shape, lambda i: (i,) + (0,) * len(shape))
    f32 = jnp.float32
    G = B // BB
    out_shapes = (
        jax.ShapeDtypeStruct((G, BB * L, D), f32),
        jax.ShapeDtypeStruct((G, BB * L, D), f32),
        jax.ShapeDtypeStruct((G, BB * L, D), f32),
        jax.ShapeDtypeStruct((G, BB * 2 * L, D), f32),
    )
    in_specs = [
        bs((BB * L * S, D)), bs((BB * L * S, 1)), bs((BB * L, D)),
        bs((BB * L, D)), bs((BB * L, D)), bs((BB * 2 * L, D)),
        bs((1, BB * L)), bs((BB, L, L)), bs((BB, 2 * L, 2 * L)),
        bs((BB, L, L)), bs((BB, L, L)),
        _full((BB * L, BB * L * S)), _full((BB * L * S, BB)),
        _full((BB, BB * L)),
        _full((D, D)), _full((1, D)), _full((D, 1)), _full((D, D)),
        _full((D, D)),
        _full((1, D)), _full((1, D)), _full((1, D)), _full((1, D)),
        _full((1, D)), _full((1, D)), _full((1, D)), _full((1, D)),
        _full((D, D)), _full((1, D)), _full((D, D)), _full((1, D)),
        _full((1, D)), _full((1, D)),
        _full((D, 3 * D)), _full((D, 3 * D)), _full((1, 3 * D)),
        _full((D, 3 * D)), _full((1, 3 * D)),
    ]
    out_specs = (bs((BB * L, D)), bs((BB * L, D)), bs((BB * L, D)),
                 bs((BB * 2 * L, D)))
    return pl.pallas_call(
        _tc_body,
        grid=(G,),
        in_specs=in_specs,
        out_specs=out_specs,
        out_shape=out_shapes,
    )(nbr, wgt3, usess, sessg, catsg, nodsg, maskf, adji, adjn, ain, aout,
      seg, ind, bsel,
      w1a, w1b, w2, w3a, w3b, la0, la1, la2, la3, ln0, ln1, ln2, ln3,
      win, bin_, wout, bout, biah, boah, wihta, wihtb, bih, whht, bhh)


def kernel(sess_idxs, global_adj_items, global_adj_wgts, urev_sess_itms,
           local_adj_itms, mask, rev_sess_itms, urev_sess_cats,
           local_adj_cats, urev_sess_nods, local_adj_nods, emb,
           la_a0, la_a1, la_a2, la_a3, lan_a0, lan_a1, lan_a2, lan_a3,
           W_in, b_in, W_out, b_out, b_iah, b_oah, w_ih, b_ih, w_hh, b_hh,
           ga_w1, ga_w2, ga_w3):
    i32 = jnp.int32
    adj_flat = global_adj_items.astype(i32).reshape(-1)
    wgt_tbl = global_adj_wgts.reshape(-1)
    urev = urev_sess_itms.reshape(-1).astype(i32)
    oth_idx = jnp.concatenate([
        urev, rev_sess_itms.reshape(-1).astype(i32),
        urev_sess_cats.reshape(-1).astype(i32),
        urev_sess_nods.reshape(-1).astype(i32)])

    wgt_flat, nbr_rows, oth_rows = _sc_gather(adj_flat, wgt_tbl, emb, urev,
                                              oth_idx)

    G = B // BB
    usess = oth_rows[:NP].reshape(G, BB * L, D)
    sessg = oth_rows[NP:2 * NP].reshape(G, BB * L, D)
    catsg = oth_rows[2 * NP:3 * NP].reshape(G, BB * L, D)
    nodsg = oth_rows[3 * NP:].reshape(G, BB * 2 * L, D)
    nbr = nbr_rows.reshape(G, BB * L * S, D)
    wgt3 = wgt_flat.reshape(G, BB * L * S, 1)
    maskf = mask.astype(jnp.float32).reshape(G, 1, BB * L)
    ain = local_adj_cats[:, :, :L].reshape(G, BB, L, L)
    aout = local_adj_cats[:, :, L:].reshape(G, BB, L, L)
    wihT = w_ih.T
    w1a = ga_w1[:D]
    w1b = ga_w1[D:].reshape(1, D)
    f32 = jnp.float32
    NR, NU = BB * L * S, BB * L
    rr = jnp.arange(NR, dtype=i32)
    ru = jnp.arange(NU, dtype=i32)
    rb = jnp.arange(BB, dtype=i32)
    seg = (ru[:, None] == rr[None, :] // S).astype(f32)
    ind = (rr[:, None] // (L * S) == rb[None, :]).astype(f32)
    bsel = (rb[:, None] == ru[None, :] // L).astype(f32)

    glob, itm, cat, nod = _tc_dense(
        nbr, wgt3, usess, sessg, catsg, nodsg, maskf,
        local_adj_itms.astype(i32).reshape(G, BB, L, L),
        local_adj_nods.astype(i32).reshape(G, BB, 2 * L, 2 * L), ain, aout,
        seg, ind, bsel,
        w1a, w1b, ga_w2, ga_w3[:D], ga_w3[D:],
        la_a0.reshape(1, D), la_a1.reshape(1, D), la_a2.reshape(1, D),
        la_a3.reshape(1, D), lan_a0.reshape(1, D), lan_a1.reshape(1, D),
        lan_a2.reshape(1, D), lan_a3.reshape(1, D),
        W_in, b_in.reshape(1, D), W_out, b_out.reshape(1, D),
        b_iah.reshape(1, D), b_oah.reshape(1, D),
        wihT[:D], wihT[D:], b_ih.reshape(1, 3 * D),
        w_hh.T, b_hh.reshape(1, 3 * D))
    return (glob.reshape(B, L, D), itm.reshape(B, L, D),
            cat.reshape(B, L, D), nod.reshape(B, 2 * L, D))


# speedup vs baseline: 2.9560x; 1.1145x over previous
import functools

import jax
import jax.numpy as jnp
from jax import lax
from jax.experimental import pallas as pl
from jax.experimental.pallas import tpu as pltpu
from jax.experimental.pallas import tpu_sc as plsc

B, L, S, D = 1024, 20, 12, 128
NP = B * L
N_NBR = NP * S
N_OTH = NP * 3 + B * 2 * L
NC, NS = 2, 16
NW = NC * NS
PPW = NP // NW
NBW = PPW * S
OPW = N_OTH // NW
CH = 128
NCH_N = NBW // CH
NCH_O = OPW // CH


def _sc_body(adj_flat, wgt_tbl, emb, urev, oth_idx,
             wgt_out, nbr_out, oth_out,
             urev_v, fidx, idxflat, wgtflat, othidx_v, rowbuf, sem, sem2):
    wid = lax.axis_index("s") * NC + lax.axis_index("c")
    pbase = wid * PPW
    pltpu.sync_copy(urev.at[pl.ds(pbase, PPW)], urev_v)

    def fidx_row(row, _):
        for t in range(8):
            k = (row * 8 + t) * 16 + lax.iota(jnp.int32, 16)
            q = k // S
            r = k - q * S
            u = plsc.load_gather(urev_v, [q])
            fidx[row, pl.ds(t * 16, 16)] = u * S + r
        return 0
    lax.fori_loop(0, NCH_N, fidx_row, 0)

    pltpu.async_copy(adj_flat.at[fidx.at[0]], idxflat.at[0], sem)
    pltpu.async_copy(wgt_tbl.at[fidx.at[0]], wgtflat.at[0], sem2)

    def elem_chunk(j, _):
        @pl.when(j < NCH_N - 1)
        def _start():
            pltpu.async_copy(adj_flat.at[fidx.at[j + 1]], idxflat.at[j + 1],
                             sem)
            pltpu.async_copy(wgt_tbl.at[fidx.at[j + 1]], wgtflat.at[j + 1],
                             sem2)
        pltpu.make_async_copy(adj_flat.at[fidx.at[j]], idxflat.at[j],
                              sem).wait()
        pltpu.make_async_copy(wgt_tbl.at[fidx.at[j]], wgtflat.at[j],
                              sem2).wait()
        return 0
    lax.fori_loop(0, NCH_N, elem_chunk, 0)
    pltpu.sync_copy(wgtflat, wgt_out.at[wid])

    def row_pipeline(idx, nch, out, out_base):
        pltpu.async_copy(emb.at[idx.at[0]], rowbuf.at[0], sem)

        def chunk(c, _):
            par = lax.rem(c, 2)
            @pl.when(c < nch - 1)
            def _start():
                pltpu.async_copy(emb.at[idx.at[c + 1]],
                                 rowbuf.at[1 - par], sem)
            pltpu.make_async_copy(emb.at[idx.at[c]], rowbuf.at[par],
                                  sem).wait()
            pltpu.sync_copy(rowbuf.at[par], out.at[pl.ds(out_base + c * CH,
                                                         CH)])
            return 0
        lax.fori_loop(0, nch, chunk, 0)

    row_pipeline(idxflat, NCH_N, nbr_out, wid * NBW)

    obase = wid * OPW

    def oidx_row(j, _):
        pltpu.sync_copy(oth_idx.at[pl.ds(obase + j * CH, CH)], othidx_v.at[j])
        return 0
    lax.fori_loop(0, NCH_O, oidx_row, 0)

    row_pipeline(othidx_v, NCH_O, oth_out, obase)


@jax.jit
def _sc_gather(adj_flat, wgt_tbl, emb, urev, oth_idx):
    mesh = plsc.VectorSubcoreMesh(core_axis_name="c", subcore_axis_name="s")
    return pl.kernel(
        _sc_body,
        out_type=(
            jax.ShapeDtypeStruct((NW, NCH_N, CH), jnp.float32),
            jax.ShapeDtypeStruct((N_NBR, D), jnp.float32),
            jax.ShapeDtypeStruct((N_OTH, D), jnp.float32),
        ),
        mesh=mesh,
        compiler_params=pltpu.CompilerParams(needs_layout_passes=False),
        scratch_types=[
            pltpu.VMEM((PPW,), jnp.int32),
            pltpu.VMEM((NCH_N, CH), jnp.int32),
            pltpu.VMEM((NCH_N, CH), jnp.int32),
            pltpu.VMEM((NCH_N, CH), jnp.float32),
            pltpu.VMEM((NCH_O, CH), jnp.int32),
            pltpu.VMEM((2, CH, D), jnp.float32),
            pltpu.SemaphoreType.DMA,
            pltpu.SemaphoreType.DMA,
        ],
    )(adj_flat, wgt_tbl, emb, urev, oth_idx)


def _leaky(x):
    return jnp.where(x >= 0, x, 0.2 * x)


BB = 8


def _blockdiag(blocks, n, dtype):
    rows = []
    for b, blk in enumerate(blocks):
        parts = []
        if b > 0:
            parts.append(jnp.zeros((n, b * n), dtype))
        parts.append(blk)
        if b < len(blocks) - 1:
            parts.append(jnp.zeros((n, (len(blocks) - 1 - b) * n), dtype))
        rows.append(jnp.concatenate(parts, axis=1))
    return jnp.concatenate(rows, axis=0)


def _latt(H, adj_ref, a0, a1, a2, a3, n):
    N = BB * n
    dn = (((1,), (1,)), ((), ()))
    e0 = _leaky(lax.dot_general(H * a0, H, dn))
    e1 = _leaky(lax.dot_general(H * a1, H, dn))
    e2 = _leaky(lax.dot_general(H * a2, H, dn))
    e3 = _leaky(lax.dot_general(H * a3, H, dn))
    ablocks = []
    for b in range(BB):
        sl = (slice(b * n, (b + 1) * n),) * 2
        adj = adj_ref[0, b]
        al = jnp.full((n, n), -9e15, jnp.float32)
        al = jnp.where(adj == 1, e0[sl], al)
        al = jnp.where(adj == 2, e1[sl], al)
        al = jnp.where(adj == 3, e2[sl], al)
        al = jnp.where(adj == 4, e3[sl], al)
        al = al - jnp.max(al, axis=-1, keepdims=True)
        ablocks.append(jnp.exp(al))
    ex = _blockdiag(ablocks, n, jnp.float32)
    return jnp.dot(ex, H) / jnp.sum(ex, axis=-1, keepdims=True)


def _tc_body(nbr_ref, wgt_ref, usess_ref, sess_ref, cats_ref, nods_ref,
             mask_ref, adji_ref, adjn_ref, ain_ref, aout_ref,
             seg_ref, ind_ref, bsel_ref,
             w1a_ref, w1b_ref, w2_ref, w3a_ref, w3b_ref,
             la0_ref, la1_ref, la2_ref, la3_ref,
             ln0_ref, ln1_ref, ln2_ref, ln3_ref,
             win_ref, bin_ref, wout_ref, bout_ref, biah_ref, boah_ref,
             wihta_ref, wihtb_ref, bih_ref, whht_ref, bhh_ref,
             glob_ref, itm_ref, cat_ref, nod_ref):
    nbr = nbr_ref[0]
    wgtc = wgt_ref[0]
    us = usess_ref[0]
    sg = sess_ref[0]
    ct = cats_ref[0]
    nd = nods_ref[0]
    m2 = mask_ref[0]
    seg = seg_ref[...]
    ind = ind_ref[...]
    bsel = bsel_ref[...]

    mbd = bsel * m2
    denom = jnp.maximum(jnp.sum(mbd, axis=-1, keepdims=True), 1.0)
    ave = jnp.dot(mbd, sg) / denom
    avebig = jnp.dot(ind, ave)

    xw = jnp.dot(nbr * avebig, w1a_ref[...]) + wgtc * w1b_ref[...]
    sc = jnp.dot(_leaky(xw), w2_ref[...])
    e = jnp.exp(sc)
    segsum = jnp.dot(seg, e)
    agg = jnp.dot(seg, e * nbr) / segsum
    glob = jax.nn.relu(jnp.dot(us, w3a_ref[...]) + jnp.dot(agg, w3b_ref[...]))
    glob_ref[0] = glob

    itm_ref[0] = _latt(us, adji_ref, la0_ref[...], la1_ref[...],
                       la2_ref[...], la3_ref[...], L)

    ain = _blockdiag([ain_ref[0, b] for b in range(BB)], L, jnp.float32)
    aout = _blockdiag([aout_ref[0, b] for b in range(BB)], L, jnp.float32)
    hv = ct
    hi = jnp.dot(ain, jnp.dot(hv, win_ref[...]) + bin_ref[...]) \
        + biah_ref[...]
    ho = jnp.dot(aout, jnp.dot(hv, wout_ref[...]) + bout_ref[...]) \
        + boah_ref[...]
    gi = jnp.dot(hi, wihta_ref[...]) + jnp.dot(ho, wihtb_ref[...]) \
        + bih_ref[...]
    gh = jnp.dot(hv, whht_ref[...]) + bhh_ref[...]
    r = jax.nn.sigmoid(gi[:, :D] + gh[:, :D])
    z = jax.nn.sigmoid(gi[:, D:2 * D] + gh[:, D:2 * D])
    n = jnp.tanh(gi[:, 2 * D:] + r * gh[:, 2 * D:])
    cat_ref[0] = n + z * (hv - n)

    nod = _latt(nd, adjn_ref, ln0_ref[...], ln1_ref[...],
                ln2_ref[...], ln3_ref[...], 2 * L)
    nod = _latt(nod, adjn_ref, ln0_ref[...], ln1_ref[...],
                ln2_ref[...], ln3_ref[...], 2 * L)
    nod_ref[0] = nod


def _full(shape):
    return pl.BlockSpec(shape, lambda i: (0,) * len(shape))


@jax.jit
def _tc_dense(nbr, wgt3, usess, sessg, catsg, nodsg, maskf, adji, adjn,
              ain, aout, seg, ind, bsel, w1a, w1b, w2, w3a, w3b,
              la0, la1, la2, la3, ln0, ln1, ln2, ln3,
              win, bin_, wout, bout, biah, boah,
              wihta, wihtb, bih, whht, bhh):
    def bs(shape):
        return pl.BlockSpec((1,) + shape, lambda i: (i,) + (0,) * len(shape))
    f32 = jnp.float32
    G = B // BB
    out_shapes = (
        jax.ShapeDtypeStruct((G, BB * L, D), f32),
        jax.ShapeDtypeStruct((G, BB * L, D), f32),
        jax.ShapeDtypeStruct((G, BB * L, D), f32),
        jax.ShapeDtypeStruct((G, BB * 2 * L, D), f32),
    )
    in_specs = [
        bs((BB * L * S, D)), bs((BB * L * S, 1)), bs((BB * L, D)),
        bs((BB * L, D)), bs((BB * L, D)), bs((BB * 2 * L, D)),
        bs((1, BB * L)), bs((BB, L, L)), bs((BB, 2 * L, 2 * L)),
        bs((BB, L, L)), bs((BB, L, L)),
        _full((BB * L, BB * L * S)), _full((BB * L * S, BB)),
        _full((BB, BB * L)),
        _full((D, D)), _full((1, D)), _full((D, 1)), _full((D, D)),
        _full((D, D)),
        _full((1, D)), _full((1, D)), _full((1, D)), _full((1, D)),
        _full((1, D)), _full((1, D)), _full((1, D)), _full((1, D)),
        _full((D, D)), _full((1, D)), _full((D, D)), _full((1, D)),
        _full((1, D)), _full((1, D)),
        _full((D, 3 * D)), _full((D, 3 * D)), _full((1, 3 * D)),
        _full((D, 3 * D)), _full((1, 3 * D)),
    ]
    out_specs = (bs((BB * L, D)), bs((BB * L, D)), bs((BB * L, D)),
                 bs((BB * 2 * L, D)))
    return pl.pallas_call(
        _tc_body,
        grid=(G,),
        in_specs=in_specs,
        out_specs=out_specs,
        out_shape=out_shapes,
    )(nbr, wgt3, usess, sessg, catsg, nodsg, maskf, adji, adjn, ain, aout,
      seg, ind, bsel,
      w1a, w1b, w2, w3a, w3b, la0, la1, la2, la3, ln0, ln1, ln2, ln3,
      win, bin_, wout, bout, biah, boah, wihta, wihtb, bih, whht, bhh)


def kernel(sess_idxs, global_adj_items, global_adj_wgts, urev_sess_itms,
           local_adj_itms, mask, rev_sess_itms, urev_sess_cats,
           local_adj_cats, urev_sess_nods, local_adj_nods, emb,
           la_a0, la_a1, la_a2, la_a3, lan_a0, lan_a1, lan_a2, lan_a3,
           W_in, b_in, W_out, b_out, b_iah, b_oah, w_ih, b_ih, w_hh, b_hh,
           ga_w1, ga_w2, ga_w3):
    i32 = jnp.int32
    adj_flat = global_adj_items.astype(i32).reshape(-1)
    wgt_tbl = global_adj_wgts.reshape(-1)
    urev = urev_sess_itms.reshape(-1).astype(i32)
    oth_idx = jnp.concatenate([
        urev, rev_sess_itms.reshape(-1).astype(i32),
        urev_sess_cats.reshape(-1).astype(i32),
        urev_sess_nods.reshape(-1).astype(i32)])

    wgt_flat, nbr_rows, oth_rows = _sc_gather(adj_flat, wgt_tbl, emb, urev,
                                              oth_idx)

    G = B // BB
    usess = oth_rows[:NP].reshape(G, BB * L, D)
    sessg = oth_rows[NP:2 * NP].reshape(G, BB * L, D)
    catsg = oth_rows[2 * NP:3 * NP].reshape(G, BB * L, D)
    nodsg = oth_rows[3 * NP:].reshape(G, BB * 2 * L, D)
    nbr = nbr_rows.reshape(G, BB * L * S, D)
    wgt3 = wgt_flat.reshape(G, BB * L * S, 1)
    maskf = mask.astype(jnp.float32).reshape(G, 1, BB * L)
    ain = local_adj_cats[:, :, :L].reshape(G, BB, L, L)
    aout = local_adj_cats[:, :, L:].reshape(G, BB, L, L)
    wihT = w_ih.T
    w1a = ga_w1[:D]
    w1b = ga_w1[D:].reshape(1, D)
    f32 = jnp.float32
    NR, NU = BB * L * S, BB * L
    rr = jnp.arange(NR, dtype=i32)
    ru = jnp.arange(NU, dtype=i32)
    rb = jnp.arange(BB, dtype=i32)
    seg = (ru[:, None] == rr[None, :] // S).astype(f32)
    ind = (rr[:, None] // (L * S) == rb[None, :]).astype(f32)
    bsel = (rb[:, None] == ru[None, :] // L).astype(f32)

    glob, itm, cat, nod = _tc_dense(
        nbr, wgt3, usess, sessg, catsg, nodsg, maskf,
        local_adj_itms.astype(i32).reshape(G, BB, L, L),
        local_adj_nods.astype(i32).reshape(G, BB, 2 * L, 2 * L), ain, aout,
        seg, ind, bsel,
        w1a, w1b, ga_w2, ga_w3[:D], ga_w3[D:],
        la_a0.reshape(1, D), la_a1.reshape(1, D), la_a2.reshape(1, D),
        la_a3.reshape(1, D), lan_a0.reshape(1, D), lan_a1.reshape(1, D),
        lan_a2.reshape(1, D), lan_a3.reshape(1, D),
        W_in, b_in.reshape(1, D), W_out, b_out.reshape(1, D),
        b_iah.reshape(1, D), b_oah.reshape(1, D),
        wihT[:D], wihT[D:], b_ih.reshape(1, 3 * D),
        w_hh.T, b_hh.reshape(1, 3 * D))
    return (glob.reshape(B, L, D), itm.reshape(B, L, D),
            cat.reshape(B, L, D), nod.reshape(B, 2 * L, D))
